# baseline (speedup 1.0000x reference)
import functools

import jax
import jax.numpy as jnp
from jax import lax
from jax.experimental import pallas as pl
from jax.experimental.pallas import tpu as pltpu
from jax.experimental.pallas import tpu_sc as plsc

_NUM_CORES = 2
_NUM_SUBCORES = 16
_NUM_WORKERS = _NUM_CORES * _NUM_SUBCORES
_LANES = 16


@functools.lru_cache(maxsize=None)
def _build(batch: int, length: int, rows_per_group: int):
    assert batch % _NUM_WORKERS == 0
    rows_per_worker = batch // _NUM_WORKERS
    assert rows_per_worker % rows_per_group == 0
    n_groups = rows_per_worker // rows_per_group
    n_chunks = length // _LANES
    R = rows_per_group

    mesh = plsc.VectorSubcoreMesh(
        core_axis_name="c", subcore_axis_name="s")

    @functools.partial(
        pl.kernel,
        out_type=jax.ShapeDtypeStruct((batch * length,), jnp.float32),
        mesh=mesh,
        compiler_params=pltpu.CompilerParams(
            needs_layout_passes=False, use_tc_tiling_on_sc=False),
        scratch_types=[
            pltpu.VMEM((length,), jnp.int32),
            pltpu.VMEM((R * length,), jnp.float32),
            pltpu.VMEM((R * length,), jnp.float32),
        ],
    )
    def hilbert_select(x_hbm, idx_hbm, out_hbm, idx_v, in_v, out_v):
        wid = lax.axis_index("s") * _NUM_CORES + lax.axis_index("c")
        base = wid * rows_per_worker * length
        pltpu.sync_copy(idx_hbm, idx_v)

        def group_body(g, carry):
            elt0 = base + g * (R * length)
            pltpu.sync_copy(x_hbm.at[pl.ds(elt0, R * length)], in_v)

            def chunk_body(ci, carry2):
                col = ci * _LANES
                iv = idx_v[pl.ds(col, _LANES)]
                for r in range(R):
                    out_v[pl.ds(r * length + col, _LANES)] = plsc.load_gather(
                        in_v, [iv + jnp.int32(r * length)])
                return carry2

            lax.fori_loop(0, n_chunks, chunk_body, 0)
            pltpu.sync_copy(out_v, out_hbm.at[pl.ds(elt0, R * length)])
            return carry

        lax.fori_loop(0, n_groups, group_body, 0)

    return hilbert_select


def kernel(x, hilbert_matrix):
    batch, length = x.shape
    idx = hilbert_matrix.reshape(-1).astype(jnp.int32)
    out = _build(batch, length, 8)(x.reshape(-1), idx)
    return out.reshape(batch, *hilbert_matrix.shape)


# speedup vs baseline: 1.4508x; 1.4508x over previous
import functools

import jax
import jax.numpy as jnp
from jax import lax
from jax.experimental import pallas as pl
from jax.experimental.pallas import tpu as pltpu
from jax.experimental.pallas import tpu_sc as plsc

_NUM_CORES = 2
_NUM_SUBCORES = 16
_NUM_WORKERS = _NUM_CORES * _NUM_SUBCORES
_LANES = 16


@functools.lru_cache(maxsize=None)
def _build(batch: int, length: int, rows_per_group: int):
    assert batch % _NUM_WORKERS == 0
    rows_per_worker = batch // _NUM_WORKERS
    assert rows_per_worker % rows_per_group == 0
    n_groups = rows_per_worker // rows_per_group
    n_chunks = length // _LANES
    R = rows_per_group
    G = R * length

    mesh = plsc.VectorSubcoreMesh(
        core_axis_name="c", subcore_axis_name="s")

    @functools.partial(
        pl.kernel,
        out_type=jax.ShapeDtypeStruct((batch * length,), jnp.float32),
        mesh=mesh,
        compiler_params=pltpu.CompilerParams(
            needs_layout_passes=False, use_tc_tiling_on_sc=False),
        scratch_types=[
            pltpu.VMEM((length,), jnp.int32),
            pltpu.VMEM((G,), jnp.float32),
            pltpu.VMEM((G,), jnp.float32),
            pltpu.VMEM((G,), jnp.float32),
            pltpu.VMEM((G,), jnp.float32),
            pltpu.SemaphoreType.DMA,
            pltpu.SemaphoreType.DMA,
            pltpu.SemaphoreType.DMA,
            pltpu.SemaphoreType.DMA,
        ],
    )
    def hilbert_select(x_hbm, idx_hbm, out_hbm, idx_v,
                       in0, in1, out0, out1, isem0, isem1, osem0, osem1):
        wid = lax.axis_index("s") * _NUM_CORES + lax.axis_index("c")
        base = wid * rows_per_worker * length
        pltpu.sync_copy(idx_hbm, idx_v)

        ins = (in0, in1)
        outs = (out0, out1)
        isems = (isem0, isem1)
        osems = (osem0, osem1)

        in_descs = [None, None]
        out_descs = [None, None]

        in_descs[0] = pltpu.async_copy(
            x_hbm.at[pl.ds(base, G)], ins[0], isems[0])

        for g in range(n_groups):
            p = g % 2
            in_descs[p].wait()
            if g + 1 < n_groups:
                in_descs[1 - p] = pltpu.async_copy(
                    x_hbm.at[pl.ds(base + (g + 1) * G, G)],
                    ins[1 - p], isems[1 - p])
            if out_descs[p] is not None:
                out_descs[p].wait()

            src = ins[p]
            dst = outs[p]

            @plsc.parallel_loop(0, n_chunks, unroll=4)
            def chunk_body(ci):
                col = ci * _LANES
                iv = idx_v[pl.ds(col, _LANES)]
                for r in range(R):
                    dst[pl.ds(r * length + col, _LANES)] = plsc.load_gather(
                        src, [iv + jnp.int32(r * length)])

            out_descs[p] = pltpu.async_copy(
                dst, out_hbm.at[pl.ds(base + g * G, G)], osems[p])

        out_descs[0].wait()
        out_descs[1].wait()

    return hilbert_select


def kernel(x, hilbert_matrix):
    batch, length = x.shape
    idx = hilbert_matrix.reshape(-1).astype(jnp.int32)
    out = _build(batch, length, 4)(x.reshape(-1), idx)
    return out.reshape(batch, *hilbert_matrix.shape)
